# initial kernel scaffold (unmeasured)
import jax
import jax.numpy as jnp
from jax import lax
from jax.experimental import pallas as pl
from jax.experimental.pallas import tpu as pltpu

N_DEV = 8
B = 2
SQ = 512
SKV = 512
E = 768
H = 8
DH = 64
BH = B * H
NEG = -1e9


def kernel(x, Wq, K_ext, V_ext, Wo):
    def body(x_ref, wq_ref, k_ref, v_ref, wo_ref, out_ref,
             qh, kfull, vfull, cth, ctx2,
             ksend, krecv, vsend, vrecv):
        my = lax.axis_index("i")
        left = lax.rem(my - 1 + N_DEV, N_DEV)
        right = lax.rem(my + 1, N_DEV)

        barrier = pltpu.get_barrier_semaphore()
        for nbr in (left, right):
            pl.semaphore_signal(barrier, inc=1, device_id=(nbr,),
                                device_id_type=pl.DeviceIdType.MESH)
        pl.semaphore_wait(barrier, 2)

        wq = wq_ref[...].astype(jnp.bfloat16)
        for b in range(B):
            xb = x_ref[b].astype(jnp.bfloat16)
            qb = lax.dot_general(xb, wq, (((1,), (0,)), ((), ())),
                                 preferred_element_type=jnp.float32)
            qb = qb.astype(jnp.bfloat16)
            for h in range(H):
                qh[b * H + h] = qb[:, h * DH:(h + 1) * DH]

        for b in range(B):
            for h in range(H):
                kfull[my, b * H + h] = k_ref[b, :, h, :].astype(jnp.bfloat16)
                vfull[my, b * H + h] = v_ref[b, :, h, :].astype(jnp.bfloat16)

        for hp in range(N_DEV - 1):
            org = lax.rem(my - hp + N_DEV, N_DEV)
            kr = pltpu.make_async_remote_copy(
                src_ref=kfull.at[org], dst_ref=kfull.at[org],
                send_sem=ksend.at[hp], recv_sem=krecv.at[hp],
                device_id=(right,), device_id_type=pl.DeviceIdType.MESH)
            vr = pltpu.make_async_remote_copy(
                src_ref=vfull.at[org], dst_ref=vfull.at[org],
                send_sem=vsend.at[hp], recv_sem=vrecv.at[hp],
                device_id=(right,), device_id_type=pl.DeviceIdType.MESH)
            kr.start()
            vr.start()
            kr.wait()
            vr.wait()

        qb2 = my * H + lax.broadcasted_iota(jnp.int32, (SQ, N_DEV * SKV), 0) // 64
        kb2 = lax.broadcasted_iota(jnp.int32, (SQ, N_DEV * SKV), 1) // 64
        keep = (qb2 == kb2) | (kb2 == 0) | (lax.rem(qb2 + kb2, 3) == 0)
        bias = jnp.where(keep, 0.0, NEG).astype(jnp.float32)

        def bh_body(bh, carry):
            q = qh[bh]
            parts = []
            for c in range(N_DEV):
                kc = kfull[c, bh]
                parts.append(lax.dot_general(
                    q, kc, (((1,), (1,)), ((), ())),
                    preferred_element_type=jnp.float32))
            s = jnp.concatenate(parts, axis=1) * 0.125 + bias
            mrow = jnp.max(s, axis=1, keepdims=True)
            p = jnp.exp(s - mrow)
            lrow = jnp.sum(p, axis=1, keepdims=True)
            pb = (p / lrow).astype(jnp.bfloat16)
            o = jnp.zeros((SQ, DH), jnp.float32)
            for c in range(N_DEV):
                o = o + lax.dot_general(
                    pb[:, c * SKV:(c + 1) * SKV], vfull[c, bh],
                    (((1,), (0,)), ((), ())),
                    preferred_element_type=jnp.float32)
            cth[bh] = o.astype(jnp.bfloat16)
            return carry

        lax.fori_loop(0, BH, bh_body, 0)

        for b in range(B):
            for h in range(H):
                ctx2[b, :, h * DH:(h + 1) * DH] = cth[b * H + h]
        wo = wo_ref[...].astype(jnp.bfloat16)
        for b in range(B):
            out_ref[b] = lax.dot_general(ctx2[b], wo, (((1,), (0,)), ((), ())),
                                         preferred_element_type=jnp.float32)

    return pl.pallas_call(
        body,
        out_shape=jax.ShapeDtypeStruct((B, SQ, E), jnp.float32),
        in_specs=[pl.BlockSpec(memory_space=pltpu.VMEM)] * 5,
        out_specs=pl.BlockSpec(memory_space=pltpu.VMEM),
        scratch_shapes=[
            pltpu.VMEM((BH, SQ, DH), jnp.bfloat16),
            pltpu.VMEM((N_DEV, BH, SKV, DH), jnp.bfloat16),
            pltpu.VMEM((N_DEV, BH, SKV, DH), jnp.bfloat16),
            pltpu.VMEM((BH, SQ, DH), jnp.bfloat16),
            pltpu.VMEM((B, SQ, H * DH), jnp.bfloat16),
            pltpu.SemaphoreType.DMA((N_DEV - 1,)),
            pltpu.SemaphoreType.DMA((N_DEV - 1,)),
            pltpu.SemaphoreType.DMA((N_DEV - 1,)),
            pltpu.SemaphoreType.DMA((N_DEV - 1,)),
        ],
        compiler_params=pltpu.CompilerParams(collective_id=0),
    )(x, Wq, K_ext, V_ext, Wo)


# baseline (device time: 214206 ns/iter reference)
import jax
import jax.numpy as jnp
from jax import lax
from jax.experimental import pallas as pl
from jax.experimental.pallas import tpu as pltpu

N_DEV = 8
B = 2
SQ = 512
SKV = 512
E = 768
H = 8
DH = 64
HD = H * DH
BH = B * H
NEG = -1e9
MINIT = -1e30


def kernel(x, Wq, K_ext, V_ext, Wo):
    def body(x_ref, wq_ref, k_ref, v_ref, wo_ref, out_ref,
             qh, kfull, vfull, khead, vhead, acc, mrow, lrow,
             ksend, krecv, vsend, vrecv):
        my = lax.axis_index("i")
        left = lax.rem(my - 1 + N_DEV, N_DEV)
        right = lax.rem(my + 1, N_DEV)

        barrier = pltpu.get_barrier_semaphore()
        for nbr in (left, right):
            pl.semaphore_signal(barrier, inc=1, device_id=(nbr,),
                                device_id_type=pl.DeviceIdType.MESH)
        pl.semaphore_wait(barrier, 2)

        kfull[my] = k_ref[...].astype(jnp.bfloat16).reshape(B, SKV, HD)
        vfull[my] = v_ref[...].astype(jnp.bfloat16).reshape(B, SKV, HD)

        wq = wq_ref[...].astype(jnp.bfloat16)
        for b in range(B):
            xb = x_ref[b].astype(jnp.bfloat16)
            qb = lax.dot_general(xb, wq, (((1,), (0,)), ((), ())),
                                 preferred_element_type=jnp.float32)
            qb = qb.astype(jnp.bfloat16)
            for h in range(H):
                qh[b * H + h] = qb[:, h * DH:(h + 1) * DH]

        mrow[...] = jnp.full((BH, 1, SQ), MINIT, jnp.float32)
        lrow[...] = jnp.zeros((BH, 1, SQ), jnp.float32)
        acc[...] = jnp.zeros((BH, DH, SQ), jnp.float32)

        def unpack(org):
            for b in range(B):
                kc = kfull[org, b]
                vc = vfull[org, b]
                for h in range(H):
                    khead[b * H + h] = kc[:, h * DH:(h + 1) * DH]
                    vhead[b * H + h] = vc[:, h * DH:(h + 1) * DH]

        unpack(my)

        qb2 = my * H + lax.broadcasted_iota(jnp.int32, (SKV, SQ), 1) // 64

        def fold_chunk(org):
            kb2 = org * H + lax.broadcasted_iota(jnp.int32, (SKV, SQ), 0) // 64
            keep = (qb2 == kb2) | (kb2 == 0) | (lax.rem(qb2 + kb2, 3) == 0)
            biasT = jnp.where(keep, 0.0, NEG).astype(jnp.float32)

            def bh_body(bh, carry):
                q = qh[bh]
                k = khead[bh]
                sT = lax.dot_general(k, q, (((1,), (1,)), ((), ())),
                                     preferred_element_type=jnp.float32)
                sT = sT * 0.125 + biasT
                m_old = mrow[bh]
                l_old = lrow[bh]
                m_new = jnp.maximum(m_old, jnp.max(sT, axis=0, keepdims=True))
                corr = jnp.exp(m_old - m_new)
                p = jnp.exp(sT - m_new)
                lrow[bh] = l_old * corr + jnp.sum(p, axis=0, keepdims=True)
                pv = lax.dot_general(vhead[bh], p.astype(jnp.bfloat16),
                                     (((0,), (0,)), ((), ())),
                                     preferred_element_type=jnp.float32)
                acc[bh] = acc[bh] * corr + pv
                mrow[bh] = m_new
                return carry

            lax.fori_loop(0, BH, bh_body, 0)

        for hp in range(N_DEV - 1):
            org = lax.rem(my - hp + 2 * N_DEV, N_DEV)
            nxt = lax.rem(my - hp - 1 + 2 * N_DEV, N_DEV)
            kr = pltpu.make_async_remote_copy(
                src_ref=kfull.at[org], dst_ref=kfull.at[org],
                send_sem=ksend.at[hp], recv_sem=krecv.at[hp],
                device_id=(right,), device_id_type=pl.DeviceIdType.MESH)
            vr = pltpu.make_async_remote_copy(
                src_ref=vfull.at[org], dst_ref=vfull.at[org],
                send_sem=vsend.at[hp], recv_sem=vrecv.at[hp],
                device_id=(right,), device_id_type=pl.DeviceIdType.MESH)
            kr.start()
            vr.start()
            fold_chunk(org)
            kr.wait()
            vr.wait()
            unpack(nxt)

        fold_chunk(lax.rem(my + 1, N_DEV))

        wo = wo_ref[...].astype(jnp.bfloat16)
        for b in range(B):
            parts = []
            for h in range(H):
                bh = b * H + h
                parts.append(acc[bh] / lrow[bh])
            ctxT = jnp.concatenate(parts, axis=0).astype(jnp.bfloat16)
            out_ref[b] = lax.dot_general(ctxT, wo, (((0,), (0,)), ((), ())),
                                         preferred_element_type=jnp.float32)

    return pl.pallas_call(
        body,
        out_shape=jax.ShapeDtypeStruct((B, SQ, E), jnp.float32),
        in_specs=[pl.BlockSpec(memory_space=pltpu.VMEM)] * 5,
        out_specs=pl.BlockSpec(memory_space=pltpu.VMEM),
        scratch_shapes=[
            pltpu.VMEM((BH, SQ, DH), jnp.bfloat16),
            pltpu.VMEM((N_DEV, B, SKV, HD), jnp.bfloat16),
            pltpu.VMEM((N_DEV, B, SKV, HD), jnp.bfloat16),
            pltpu.VMEM((BH, SKV, DH), jnp.bfloat16),
            pltpu.VMEM((BH, SKV, DH), jnp.bfloat16),
            pltpu.VMEM((BH, DH, SQ), jnp.float32),
            pltpu.VMEM((BH, 1, SQ), jnp.float32),
            pltpu.VMEM((BH, 1, SQ), jnp.float32),
            pltpu.SemaphoreType.DMA((N_DEV - 1,)),
            pltpu.SemaphoreType.DMA((N_DEV - 1,)),
            pltpu.SemaphoreType.DMA((N_DEV - 1,)),
            pltpu.SemaphoreType.DMA((N_DEV - 1,)),
        ],
        compiler_params=pltpu.CompilerParams(
            collective_id=0, vmem_limit_bytes=100 * 1024 * 1024),
    )(x, Wq, K_ext, V_ext, Wo)


# device time: 135950 ns/iter; 1.5756x vs baseline; 1.5756x over previous
import jax
import jax.numpy as jnp
from jax import lax
from jax.experimental import pallas as pl
from jax.experimental.pallas import tpu as pltpu

N_DEV = 8
B = 2
SQ = 512
SKV = 512
HKV = SKV // 2
E = 768
H = 8
DH = 64
HD = H * DH
BH = B * H
NEG = -1e9
MINIT = -1e30


def kernel(x, Wq, K_ext, V_ext, Wo):
    def body(x_ref, wq_ref, k_ref, v_ref, wo_ref, out_ref,
             qh, kfull, vfull, khead, vhead, acc, mrow, lrow,
             ksR, krR, vsR, vrR, ksL, krL, vsL, vrL):
        my = lax.axis_index("i")

        def pr(t):
            return jnp.where(t < 4, t, 11 - t)

        r = pr(my)
        right = pr(lax.rem(r + 1, N_DEV))
        left = pr(lax.rem(r - 1 + N_DEV, N_DEV))

        barrier = pltpu.get_barrier_semaphore()
        for nbr in (left, right):
            pl.semaphore_signal(barrier, inc=1, device_id=(nbr,),
                                device_id_type=pl.DeviceIdType.MESH)
        pl.semaphore_wait(barrier, 2)

        kfull[my] = k_ref[...].astype(jnp.bfloat16).reshape(B, SKV, HD)
        vfull[my] = v_ref[...].astype(jnp.bfloat16).reshape(B, SKV, HD)

        wq = wq_ref[...].astype(jnp.bfloat16)
        for b in range(B):
            xb = x_ref[b].astype(jnp.bfloat16)
            qb = lax.dot_general(xb, wq, (((1,), (0,)), ((), ())),
                                 preferred_element_type=jnp.float32)
            qb = qb.astype(jnp.bfloat16)
            for h in range(H):
                qh[b * H + h] = qb[:, h * DH:(h + 1) * DH]

        mrow[...] = jnp.full((BH, 1, SQ), MINIT, jnp.float32)
        lrow[...] = jnp.zeros((BH, 1, SQ), jnp.float32)
        acc[...] = jnp.zeros((BH, DH, SQ), jnp.float32)

        def unpack(org_a, org_b):
            for b in range(B):
                ka = kfull[org_a, b]
                va = vfull[org_a, b]
                kb_ = kfull[org_b, b]
                vb_ = vfull[org_b, b]
                for h in range(H):
                    sl = slice(h * DH, (h + 1) * DH)
                    khead[b * H + h, 0:HKV] = ka[0:HKV, sl]
                    vhead[b * H + h, 0:HKV] = va[0:HKV, sl]
                    khead[b * H + h, HKV:SKV] = kb_[HKV:SKV, sl]
                    vhead[b * H + h, HKV:SKV] = vb_[HKV:SKV, sl]

        unpack(my, my)

        qb2 = my * H + lax.broadcasted_iota(jnp.int32, (SKV, SQ), 1) // 64
        row_i = lax.broadcasted_iota(jnp.int32, (SKV, SQ), 0)

        def fold(org_a, org_b):
            org_row = jnp.where(row_i < HKV, org_a, org_b)
            kb2 = org_row * H + row_i // 64
            keep = (qb2 == kb2) | (kb2 == 0) | (lax.rem(qb2 + kb2, 3) == 0)
            biasT = jnp.where(keep, 0.0, NEG).astype(jnp.float32)

            def bh_body(bh, carry):
                q = qh[bh]
                k = khead[bh]
                sT = lax.dot_general(k, q, (((1,), (1,)), ((), ())),
                                     preferred_element_type=jnp.float32)
                sT = sT * 0.125 + biasT
                m_old = mrow[bh]
                l_old = lrow[bh]
                m_new = jnp.maximum(m_old, jnp.max(sT, axis=0, keepdims=True))
                corr = jnp.exp(m_old - m_new)
                p = jnp.exp(sT - m_new)
                lrow[bh] = l_old * corr + jnp.sum(p, axis=0, keepdims=True)
                pv = lax.dot_general(vhead[bh], p.astype(jnp.bfloat16),
                                     (((0,), (0,)), ((), ())),
                                     preferred_element_type=jnp.float32)
                acc[bh] = acc[bh] * corr + pv
                mrow[bh] = m_new
                return carry

            lax.fori_loop(0, BH, bh_body, 0)

        for hp in range(N_DEV - 1):
            org_a = pr(lax.rem(r - hp + 2 * N_DEV, N_DEV))
            org_b = pr(lax.rem(r + hp, N_DEV))
            nxt_a = pr(lax.rem(r - hp - 1 + 2 * N_DEV, N_DEV))
            nxt_b = pr(lax.rem(r + hp + 1, N_DEV))
            rdmas = []
            for (full, ss, rs) in ((kfull, ksR, krR), (vfull, vsR, vrR)):
                rdmas.append(pltpu.make_async_remote_copy(
                    src_ref=full.at[org_a, :, pl.ds(0, HKV), :],
                    dst_ref=full.at[org_a, :, pl.ds(0, HKV), :],
                    send_sem=ss.at[hp], recv_sem=rs.at[hp],
                    device_id=(right,), device_id_type=pl.DeviceIdType.MESH))
            for (full, ss, rs) in ((kfull, ksL, krL), (vfull, vsL, vrL)):
                rdmas.append(pltpu.make_async_remote_copy(
                    src_ref=full.at[org_b, :, pl.ds(HKV, HKV), :],
                    dst_ref=full.at[org_b, :, pl.ds(HKV, HKV), :],
                    send_sem=ss.at[hp], recv_sem=rs.at[hp],
                    device_id=(left,), device_id_type=pl.DeviceIdType.MESH))
            for rd in rdmas:
                rd.start()
            fold(org_a, org_b)
            for rd in rdmas:
                rd.wait()
            unpack(nxt_a, nxt_b)

        fold(pr(lax.rem(r + 1, N_DEV)), pr(lax.rem(r - 1 + N_DEV, N_DEV)))

        wo = wo_ref[...].astype(jnp.bfloat16)
        for b in range(B):
            parts = []
            for h in range(H):
                bh = b * H + h
                parts.append(acc[bh] / lrow[bh])
            ctxT = jnp.concatenate(parts, axis=0).astype(jnp.bfloat16)
            out_ref[b] = lax.dot_general(ctxT, wo, (((0,), (0,)), ((), ())),
                                         preferred_element_type=jnp.float32)

    return pl.pallas_call(
        body,
        out_shape=jax.ShapeDtypeStruct((B, SQ, E), jnp.float32),
        in_specs=[pl.BlockSpec(memory_space=pltpu.VMEM)] * 5,
        out_specs=pl.BlockSpec(memory_space=pltpu.VMEM),
        scratch_shapes=[
            pltpu.VMEM((BH, SQ, DH), jnp.bfloat16),
            pltpu.VMEM((N_DEV, B, SKV, HD), jnp.bfloat16),
            pltpu.VMEM((N_DEV, B, SKV, HD), jnp.bfloat16),
            pltpu.VMEM((BH, SKV, DH), jnp.bfloat16),
            pltpu.VMEM((BH, SKV, DH), jnp.bfloat16),
            pltpu.VMEM((BH, DH, SQ), jnp.float32),
            pltpu.VMEM((BH, 1, SQ), jnp.float32),
            pltpu.VMEM((BH, 1, SQ), jnp.float32),
        ] + [pltpu.SemaphoreType.DMA((N_DEV - 1,))] * 8,
        compiler_params=pltpu.CompilerParams(
            collective_id=0, vmem_limit_bytes=100 * 1024 * 1024),
    )(x, Wq, K_ext, V_ext, Wo)


# device time: 133293 ns/iter; 1.6070x vs baseline; 1.0199x over previous
import jax
import jax.numpy as jnp
from jax import lax
from jax.experimental import pallas as pl
from jax.experimental.pallas import tpu as pltpu

N_DEV = 8
B = 2
SQ = 512
SKV = 512
HKV = SKV // 2
E = 768
H = 8
DH = 64
HD = H * DH
BH = B * H
NEG = -1e9
MINIT = -1e30


def kernel(x, Wq, K_ext, V_ext, Wo):
    def body(x_ref, wq_ref, k_ref, v_ref, wo_ref, out_ref,
             qh, kfull, vfull, khead, vhead, acc, lrow,
             ksR, krR, vsR, vrR, ksL, krL, vsL, vrL):
        my = lax.axis_index("i")

        def pr(t):
            return jnp.where(t < 4, t, 11 - t)

        r = pr(my)
        right = pr(lax.rem(r + 1, N_DEV))
        left = pr(lax.rem(r - 1 + N_DEV, N_DEV))

        barrier = pltpu.get_barrier_semaphore()
        for nbr in (left, right):
            pl.semaphore_signal(barrier, inc=1, device_id=(nbr,),
                                device_id_type=pl.DeviceIdType.MESH)
        pl.semaphore_wait(barrier, 2)

        kfull[my] = k_ref[...].astype(jnp.bfloat16).reshape(B, SKV, HD)
        vfull[my] = v_ref[...].astype(jnp.bfloat16).reshape(B, SKV, HD)

        wq = wq_ref[...].astype(jnp.bfloat16)
        for b in range(B):
            xb = x_ref[b].astype(jnp.bfloat16)
            qb = lax.dot_general(xb, wq, (((1,), (0,)), ((), ())),
                                 preferred_element_type=jnp.float32)
            qb = (qb * (0.125 * 1.4426950408889634)).astype(jnp.bfloat16)
            for h in range(H):
                qh[b * H + h] = qb[:, h * DH:(h + 1) * DH]

        lrow[...] = jnp.zeros((BH, 1, SQ), jnp.float32)
        acc[...] = jnp.zeros((BH, DH, SQ), jnp.float32)

        def unpack(org_a, org_b):
            for b in range(B):
                ka = kfull[org_a, b]
                va = vfull[org_a, b]
                kb_ = kfull[org_b, b]
                vb_ = vfull[org_b, b]
                for h in range(H):
                    sl = slice(h * DH, (h + 1) * DH)
                    khead[b * H + h, 0:HKV] = ka[0:HKV, sl]
                    vhead[b * H + h, 0:HKV] = va[0:HKV, sl]
                    khead[b * H + h, HKV:SKV] = kb_[HKV:SKV, sl]
                    vhead[b * H + h, HKV:SKV] = vb_[HKV:SKV, sl]

        unpack(my, my)

        qb2 = my * H + lax.broadcasted_iota(jnp.int32, (SKV, SQ), 1) // 64
        row_i = lax.broadcasted_iota(jnp.int32, (SKV, SQ), 0)

        def fold(org_a, org_b):
            org_row = jnp.where(row_i < HKV, org_a, org_b)
            kb2 = org_row * H + row_i // 64
            keep = (qb2 == kb2) | (kb2 == 0) | (lax.rem(qb2 + kb2, 3) == 0)
            biasT = jnp.where(keep, 0.0, NEG).astype(jnp.float32)

            def bh_body(bh, carry):
                q = qh[bh]
                k = khead[bh]
                sT = lax.dot_general(k, q, (((1,), (1,)), ((), ())),
                                     preferred_element_type=jnp.float32)
                p = jnp.exp2(sT + biasT)
                lrow[bh] = lrow[bh] + jnp.sum(p, axis=0, keepdims=True)
                pv = lax.dot_general(vhead[bh], p.astype(jnp.bfloat16),
                                     (((0,), (0,)), ((), ())),
                                     preferred_element_type=jnp.float32)
                acc[bh] = acc[bh] + pv
                return carry

            lax.fori_loop(0, BH, bh_body, 0)

        for hp in range(N_DEV - 1):
            org_a = pr(lax.rem(r - hp + 2 * N_DEV, N_DEV))
            org_b = pr(lax.rem(r + hp, N_DEV))
            nxt_a = pr(lax.rem(r - hp - 1 + 2 * N_DEV, N_DEV))
            nxt_b = pr(lax.rem(r + hp + 1, N_DEV))
            rdmas = []
            for (full, ss, rs) in ((kfull, ksR, krR), (vfull, vsR, vrR)):
                rdmas.append(pltpu.make_async_remote_copy(
                    src_ref=full.at[org_a, :, pl.ds(0, HKV), :],
                    dst_ref=full.at[org_a, :, pl.ds(0, HKV), :],
                    send_sem=ss.at[hp], recv_sem=rs.at[hp],
                    device_id=(right,), device_id_type=pl.DeviceIdType.MESH))
            for (full, ss, rs) in ((kfull, ksL, krL), (vfull, vsL, vrL)):
                rdmas.append(pltpu.make_async_remote_copy(
                    src_ref=full.at[org_b, :, pl.ds(HKV, HKV), :],
                    dst_ref=full.at[org_b, :, pl.ds(HKV, HKV), :],
                    send_sem=ss.at[hp], recv_sem=rs.at[hp],
                    device_id=(left,), device_id_type=pl.DeviceIdType.MESH))
            for rd in rdmas:
                rd.start()
            fold(org_a, org_b)
            for rd in rdmas:
                rd.wait()
            unpack(nxt_a, nxt_b)

        fold(pr(lax.rem(r + 1, N_DEV)), pr(lax.rem(r - 1 + N_DEV, N_DEV)))

        wo = wo_ref[...].astype(jnp.bfloat16)
        for b in range(B):
            parts = []
            for h in range(H):
                bh = b * H + h
                parts.append(acc[bh] / lrow[bh])
            ctxT = jnp.concatenate(parts, axis=0).astype(jnp.bfloat16)
            out_ref[b] = lax.dot_general(ctxT, wo, (((0,), (0,)), ((), ())),
                                         preferred_element_type=jnp.float32)

    return pl.pallas_call(
        body,
        out_shape=jax.ShapeDtypeStruct((B, SQ, E), jnp.float32),
        in_specs=[pl.BlockSpec(memory_space=pltpu.VMEM)] * 5,
        out_specs=pl.BlockSpec(memory_space=pltpu.VMEM),
        scratch_shapes=[
            pltpu.VMEM((BH, SQ, DH), jnp.bfloat16),
            pltpu.VMEM((N_DEV, B, SKV, HD), jnp.bfloat16),
            pltpu.VMEM((N_DEV, B, SKV, HD), jnp.bfloat16),
            pltpu.VMEM((BH, SKV, DH), jnp.bfloat16),
            pltpu.VMEM((BH, SKV, DH), jnp.bfloat16),
            pltpu.VMEM((BH, DH, SQ), jnp.float32),
            pltpu.VMEM((BH, 1, SQ), jnp.float32),
        ] + [pltpu.SemaphoreType.DMA((N_DEV - 1,))] * 8,
        compiler_params=pltpu.CompilerParams(
            collective_id=0, vmem_limit_bytes=100 * 1024 * 1024),
    )(x, Wq, K_ext, V_ext, Wo)


# device time: 126140 ns/iter; 1.6982x vs baseline; 1.0567x over previous
import jax
import jax.numpy as jnp
from jax import lax
from jax.experimental import pallas as pl
from jax.experimental.pallas import tpu as pltpu

N_DEV = 8
B = 2
SQ = 512
SKV = 512
HKV = SKV // 2
E = 768
H = 8
DH = 64
HD = H * DH
BH = B * H
NEG = -1e9
QSCALE = 0.125 * 1.4426950408889634


def kernel(x, Wq, K_ext, V_ext, Wo):
    def body(x_ref, wq_ref, k_ref, v_ref, wo_ref, out_ref,
             qh, kvfull, khead, vhead, acc, lrow,
             sR, rR, sL, rL):
        my = lax.axis_index("i")

        def pr(t):
            return jnp.where(t < 4, t, 11 - t)

        r = pr(my)
        right = pr(lax.rem(r + 1, N_DEV))
        left = pr(lax.rem(r - 1 + N_DEV, N_DEV))

        barrier = pltpu.get_barrier_semaphore()
        for nbr in (left, right):
            pl.semaphore_signal(barrier, inc=1, device_id=(nbr,),
                                device_id_type=pl.DeviceIdType.MESH)

        kvfull[my, 0] = k_ref[...].astype(jnp.bfloat16).reshape(B, SKV, HD)
        kvfull[my, 1] = v_ref[...].astype(jnp.bfloat16).reshape(B, SKV, HD)

        wq = wq_ref[...].astype(jnp.bfloat16)
        for b in range(B):
            xb = x_ref[b].astype(jnp.bfloat16)
            qb = lax.dot_general(xb, wq, (((1,), (0,)), ((), ())),
                                 preferred_element_type=jnp.float32)
            qb = (qb * QSCALE).astype(jnp.bfloat16)
            for h in range(H):
                qh[b * H + h] = qb[:, h * DH:(h + 1) * DH]

        lrow[...] = jnp.zeros((BH, 1, SQ), jnp.float32)
        acc[...] = jnp.zeros((BH, DH, SQ), jnp.float32)

        def unpack(org_a, org_b):
            for b in range(B):
                ka = kvfull[org_a, 0, b]
                va = kvfull[org_a, 1, b]
                kb_ = kvfull[org_b, 0, b]
                vb_ = kvfull[org_b, 1, b]
                for h in range(H):
                    sl = slice(h * DH, (h + 1) * DH)
                    khead[b * H + h, 0:HKV] = ka[0:HKV, sl]
                    vhead[b * H + h, 0:HKV] = va[0:HKV, sl]
                    khead[b * H + h, HKV:SKV] = kb_[HKV:SKV, sl]
                    vhead[b * H + h, HKV:SKV] = vb_[HKV:SKV, sl]

        unpack(my, my)

        pl.semaphore_wait(barrier, 2)

        qb2 = my * H + lax.broadcasted_iota(jnp.int32, (SKV, SQ), 1) // 64
        row_i = lax.broadcasted_iota(jnp.int32, (SKV, SQ), 0)

        def fold(org_a, org_b):
            org_row = jnp.where(row_i < HKV, org_a, org_b)
            kb2 = org_row * H + row_i // 64
            keep = (qb2 == kb2) | (kb2 == 0) | (lax.rem(qb2 + kb2, 3) == 0)
            biasT = jnp.where(keep, 0.0, NEG).astype(jnp.float32)

            def bh_body(bh, carry):
                q = qh[bh]
                k = khead[bh]
                sT = lax.dot_general(k, q, (((1,), (1,)), ((), ())),
                                     preferred_element_type=jnp.float32)
                p = jnp.exp2(sT + biasT)
                lrow[bh] = lrow[bh] + jnp.sum(p, axis=0, keepdims=True)
                pv = lax.dot_general(vhead[bh], p.astype(jnp.bfloat16),
                                     (((0,), (0,)), ((), ())),
                                     preferred_element_type=jnp.float32)
                acc[bh] = acc[bh] + pv
                return carry

            lax.fori_loop(0, BH, bh_body, 0)

        def hop_rdmas(hp, org_a, org_b):
            ra = pltpu.make_async_remote_copy(
                src_ref=kvfull.at[org_a, :, :, pl.ds(0, HKV), :],
                dst_ref=kvfull.at[org_a, :, :, pl.ds(0, HKV), :],
                send_sem=sR.at[hp], recv_sem=rR.at[hp],
                device_id=(right,), device_id_type=pl.DeviceIdType.MESH)
            rb = pltpu.make_async_remote_copy(
                src_ref=kvfull.at[org_b, :, :, pl.ds(HKV, HKV), :],
                dst_ref=kvfull.at[org_b, :, :, pl.ds(HKV, HKV), :],
                send_sem=sL.at[hp], recv_sem=rL.at[hp],
                device_id=(left,), device_id_type=pl.DeviceIdType.MESH)
            return ra, rb

        orgs = []
        for hp in range(N_DEV):
            orgs.append((pr(lax.rem(r - hp + 2 * N_DEV, N_DEV)),
                         pr(lax.rem(r + hp, N_DEV))))

        ra, rb = hop_rdmas(0, *orgs[0])
        ra.start()
        rb.start()
        fold(*orgs[0])
        ra.wait_recv()
        rb.wait_recv()
        for hp in range(1, N_DEV - 1):
            ra, rb = hop_rdmas(hp, *orgs[hp])
            ra.start()
            rb.start()
            unpack(*orgs[hp])
            fold(*orgs[hp])
            ra.wait_recv()
            rb.wait_recv()
        unpack(*orgs[N_DEV - 1])
        fold(*orgs[N_DEV - 1])

        for hp in range(N_DEV - 1):
            ra, rb = hop_rdmas(hp, *orgs[hp])
            ra.wait_send()
            rb.wait_send()

        wo = wo_ref[...].astype(jnp.bfloat16)
        for b in range(B):
            parts = []
            for h in range(H):
                bh = b * H + h
                parts.append(acc[bh] / lrow[bh])
            ctxT = jnp.concatenate(parts, axis=0).astype(jnp.bfloat16)
            out_ref[b] = lax.dot_general(ctxT, wo, (((0,), (0,)), ((), ())),
                                         preferred_element_type=jnp.float32)

    return pl.pallas_call(
        body,
        out_shape=jax.ShapeDtypeStruct((B, SQ, E), jnp.float32),
        in_specs=[pl.BlockSpec(memory_space=pltpu.VMEM)] * 5,
        out_specs=pl.BlockSpec(memory_space=pltpu.VMEM),
        scratch_shapes=[
            pltpu.VMEM((BH, SQ, DH), jnp.bfloat16),
            pltpu.VMEM((N_DEV, 2, B, SKV, HD), jnp.bfloat16),
            pltpu.VMEM((BH, SKV, DH), jnp.bfloat16),
            pltpu.VMEM((BH, SKV, DH), jnp.bfloat16),
            pltpu.VMEM((BH, DH, SQ), jnp.float32),
            pltpu.VMEM((BH, 1, SQ), jnp.float32),
        ] + [pltpu.SemaphoreType.DMA((N_DEV - 1,))] * 4,
        compiler_params=pltpu.CompilerParams(
            collective_id=0, vmem_limit_bytes=100 * 1024 * 1024),
    )(x, Wq, K_ext, V_ext, Wo)


# device time: 117998 ns/iter; 1.8153x vs baseline; 1.0690x over previous
import jax
import jax.numpy as jnp
from jax import lax
from jax.experimental import pallas as pl
from jax.experimental.pallas import tpu as pltpu

N_DEV = 8
B = 2
SQ = 512
SKV = 512
HKV = SKV // 2
E = 768
H = 8
DH = 64
HD = H * DH
BH = B * H
NEG = -1e9
QSCALE = 0.125 * 1.4426950408889634


def kernel(x, Wq, K_ext, V_ext, Wo):
    def body(x_ref, wq_ref, k_ref, v_ref, wo_ref, out_ref,
             qh, kvfull, khead, vhead, acc, lrow,
             sA1, rA1, sA2, rA2, sB1, rB1, sB2, rB2):
        my = lax.axis_index("i")

        def pr(t):
            return jnp.where(t < 4, t, 11 - t)

        r = pr(my)
        right = pr(lax.rem(r + 1, N_DEV))
        left = pr(lax.rem(r - 1 + N_DEV, N_DEV))

        barrier = pltpu.get_barrier_semaphore()
        for nbr in (left, right):
            pl.semaphore_signal(barrier, inc=1, device_id=(nbr,),
                                device_id_type=pl.DeviceIdType.MESH)

        kvfull[my, 0] = k_ref[...].astype(jnp.bfloat16).reshape(B, SKV, HD)
        kvfull[my, 1] = v_ref[...].astype(jnp.bfloat16).reshape(B, SKV, HD)

        wq = wq_ref[...].astype(jnp.bfloat16)
        for b in range(B):
            xb = x_ref[b].astype(jnp.bfloat16)
            qb = lax.dot_general(xb, wq, (((1,), (0,)), ((), ())),
                                 preferred_element_type=jnp.float32)
            qb = (qb * QSCALE).astype(jnp.bfloat16)
            for h in range(H):
                qh[b * H + h] = qb[:, h * DH:(h + 1) * DH]

        lrow[...] = jnp.zeros((BH, 1, SQ), jnp.float32)
        acc[...] = jnp.zeros((BH, DH, SQ), jnp.float32)

        def unpack(org_a, org_b):
            for b in range(B):
                ka = kvfull[org_a, 0, b]
                va = kvfull[org_a, 1, b]
                kb_ = kvfull[org_b, 0, b]
                vb_ = kvfull[org_b, 1, b]
                for h in range(H):
                    sl = slice(h * DH, (h + 1) * DH)
                    khead[b * H + h, 0:HKV] = ka[0:HKV, sl]
                    vhead[b * H + h, 0:HKV] = va[0:HKV, sl]
                    khead[b * H + h, HKV:SKV] = kb_[HKV:SKV, sl]
                    vhead[b * H + h, HKV:SKV] = vb_[HKV:SKV, sl]

        unpack(my, my)

        pl.semaphore_wait(barrier, 2)

        qb2 = my * H + lax.broadcasted_iota(jnp.int32, (SKV, SQ), 1) // 64
        row_i = lax.broadcasted_iota(jnp.int32, (SKV, SQ), 0)

        def fold(org_a, org_b):
            org_row = jnp.where(row_i < HKV, org_a, org_b)
            kb2 = org_row * H + row_i // 64
            keep = (qb2 == kb2) | (kb2 == 0) | (lax.rem(qb2 + kb2, 3) == 0)
            biasT = jnp.where(keep, 0.0, NEG).astype(jnp.float32)

            def bh_body(bh, carry):
                q = qh[bh]
                k = khead[bh]
                sT = lax.dot_general(k, q, (((1,), (1,)), ((), ())),
                                     preferred_element_type=jnp.float32)
                p = jnp.exp2(sT + biasT)
                lrow[bh] = lrow[bh] + jnp.sum(p, axis=0, keepdims=True)
                pv = lax.dot_general(vhead[bh], p.astype(jnp.bfloat16),
                                     (((0,), (0,)), ((), ())),
                                     preferred_element_type=jnp.float32)
                acc[bh] = acc[bh] + pv
                return carry

            lax.fori_loop(0, BH, bh_body, 0)

        QKV = HKV // 2

        def hop_rdmas(hp, org_a, org_b):
            out = []
            for qi, (org, off, dev, ss, rs) in enumerate((
                    (org_a, 0, right, sA1, rA1),
                    (org_a, QKV, right, sA2, rA2),
                    (org_b, HKV, left, sB1, rB1),
                    (org_b, HKV + QKV, left, sB2, rB2))):
                out.append(pltpu.make_async_remote_copy(
                    src_ref=kvfull.at[org, :, :, pl.ds(off, QKV), :],
                    dst_ref=kvfull.at[org, :, :, pl.ds(off, QKV), :],
                    send_sem=ss.at[hp], recv_sem=rs.at[hp],
                    device_id=(dev,), device_id_type=pl.DeviceIdType.MESH))
            return out

        orgs = []
        for hp in range(N_DEV):
            orgs.append((pr(lax.rem(r - hp + 2 * N_DEV, N_DEV)),
                         pr(lax.rem(r + hp, N_DEV))))

        a1, a2, b1, b2 = hop_rdmas(0, *orgs[0])
        a1.start()
        b1.start()
        a2.start()
        b2.start()
        fold(*orgs[0])
        a1.wait_recv()
        b1.wait_recv()
        prev_a2, prev_b2 = a2, b2
        for hp in range(1, N_DEV - 1):
            a1, a2, b1, b2 = hop_rdmas(hp, *orgs[hp])
            a1.start()
            b1.start()
            prev_a2.wait_recv()
            prev_b2.wait_recv()
            a2.start()
            b2.start()
            unpack(*orgs[hp])
            fold(*orgs[hp])
            a1.wait_recv()
            b1.wait_recv()
            prev_a2, prev_b2 = a2, b2
        prev_a2.wait_recv()
        prev_b2.wait_recv()
        unpack(*orgs[N_DEV - 1])
        fold(*orgs[N_DEV - 1])

        for hp in range(N_DEV - 1):
            for rd in hop_rdmas(hp, *orgs[hp]):
                rd.wait_send()

        wo = wo_ref[...].astype(jnp.bfloat16)
        for b in range(B):
            parts = []
            for h in range(H):
                bh = b * H + h
                parts.append(acc[bh] / lrow[bh])
            ctxT = jnp.concatenate(parts, axis=0).astype(jnp.bfloat16)
            out_ref[b] = lax.dot_general(ctxT, wo, (((0,), (0,)), ((), ())),
                                         preferred_element_type=jnp.float32)

    return pl.pallas_call(
        body,
        out_shape=jax.ShapeDtypeStruct((B, SQ, E), jnp.float32),
        in_specs=[pl.BlockSpec(memory_space=pltpu.VMEM)] * 5,
        out_specs=pl.BlockSpec(memory_space=pltpu.VMEM),
        scratch_shapes=[
            pltpu.VMEM((BH, SQ, DH), jnp.bfloat16),
            pltpu.VMEM((N_DEV, 2, B, SKV, HD), jnp.bfloat16),
            pltpu.VMEM((BH, SKV, DH), jnp.bfloat16),
            pltpu.VMEM((BH, SKV, DH), jnp.bfloat16),
            pltpu.VMEM((BH, DH, SQ), jnp.float32),
            pltpu.VMEM((BH, 1, SQ), jnp.float32),
        ] + [pltpu.SemaphoreType.DMA((N_DEV - 1,))] * 8,
        compiler_params=pltpu.CompilerParams(
            collective_id=0, vmem_limit_bytes=100 * 1024 * 1024),
    )(x, Wq, K_ext, V_ext, Wo)
